# baseline (speedup 1.0000x reference)
import functools

import jax
import jax.numpy as jnp
from jax import lax
from jax.experimental import pallas as pl
from jax.experimental.pallas import tpu as pltpu
from jax.experimental.pallas import tpu_sc as plsc

M = 25000
N = 100000
K = 3
C = 128

NC = 2
NS = 16
NW = NC * NS
ROWS_PER_W = N // NW
B = 125
BP = 128
NBLK = ROWS_PER_W // B
LANES = 16
CCHUNKS = C // LANES
KBP = K * BP
FULLG = B // LANES
TAIL = B - FULLG * LANES
PIB = lax.GatherScatterMode.PROMISE_IN_BOUNDS
DNUMS = lax.GatherDimensionNumbers(
    offset_dims=(), collapsed_slice_dims=(0,), start_index_map=(0,)
)


def _unpool_body(
    table, w_hbm, idx_hbm, out_hbm,
    idxv0, idxv1, wv0, wv1, rows0, rows1, outv,
    gsem0, gsem1, osem,
):
    wid = lax.axis_index("s") * NC + lax.axis_index("c")
    idxv = (idxv0, idxv1)
    wv = (wv0, wv1)
    rows = (rows0, rows1)
    gsem = (gsem0, gsem1)

    def prefetch(g, s):
        blkid = wid * NBLK + g
        pltpu.sync_copy(idx_hbm.at[pl.ds(blkid * KBP, KBP)], idxv[s])
        pltpu.sync_copy(w_hbm.at[pl.ds(blkid * KBP, KBP)], wv[s])
        for k in range(K):
            pltpu.async_copy(
                table.at[idxv[s].at[pl.ds(k * BP, B)]], rows[s].at[k], gsem[s]
            )

    def wait_gathers(s):
        for k in range(K):
            pltpu.make_async_copy(
                table.at[idxv[s].at[pl.ds(k * BP, B)]], rows[s].at[k], gsem[s]
            ).wait()

    def wait_out():
        pltpu.make_async_copy(
            outv, out_hbm.at[pl.ds(0, B * C)], osem
        ).wait()

    def fire_out(g):
        pltpu.async_copy(
            outv,
            out_hbm.at[pl.ds((wid * ROWS_PER_W + g * B) * C, B * C)],
            osem,
        )

    def compute(s):
        rs = rows[s]
        ws = wv[s]
        ov = outv

        def make_rowfn(b0, wvecs):
            def rowfn(lane, c1):
                b = b0 + lane
                lv = jnp.full((LANES, 1), lane, dtype=jnp.int32)
                w0 = lax.gather(wvecs[0], lv, DNUMS, (1,), mode=PIB)
                w1 = lax.gather(wvecs[1], lv, DNUMS, (1,), mode=PIB)
                w2 = lax.gather(wvecs[2], lv, DNUMS, (1,), mode=PIB)
                for c in range(CCHUNKS):
                    sl = pl.ds(c * LANES, LANES)
                    ov[pl.ds(b * C + c * LANES, LANES)] = (
                        w0 * rs[0, b, sl]
                        + w1 * rs[1, b, sl]
                        + w2 * rs[2, b, sl]
                    )
                return c1

            return rowfn

        def grp(g16, c2):
            b0 = g16 * LANES
            wvecs = [ws[pl.ds(k * BP + b0, LANES)] for k in range(K)]
            lax.fori_loop(0, LANES, make_rowfn(b0, wvecs), 0, unroll=8)
            return c2

        lax.fori_loop(0, FULLG, grp, 0, unroll=1)
        b0t = FULLG * LANES
        wvecs_t = [ws[pl.ds(k * BP + b0t, LANES)] for k in range(K)]
        lax.fori_loop(0, TAIL, make_rowfn(b0t, wvecs_t), 0, unroll=1)

    prefetch(0, 0)
    prefetch(1, 1)
    wait_gathers(0)
    compute(0)
    fire_out(0)
    prefetch(2, 0)
    wait_out()
    wait_gathers(1)
    compute(1)
    fire_out(1)

    def pair(p, carry):
        g = 2 * p
        prefetch(g + 1, 1)
        wait_out()
        wait_gathers(0)
        compute(0)
        fire_out(g)
        prefetch(g + 2, 0)
        wait_out()
        wait_gathers(1)
        compute(1)
        fire_out(g + 1)
        return carry

    lax.fori_loop(1, NBLK // 2, pair, 0, unroll=1)

    wait_out()
    wait_gathers(0)
    compute(0)
    fire_out(NBLK - 1)
    wait_out()


@jax.jit
def _unpool(table, w_arr, idx_arr):
    mesh = plsc.VectorSubcoreMesh(core_axis_name="c", subcore_axis_name="s")
    f = functools.partial(
        pl.kernel,
        mesh=mesh,
        out_type=jax.ShapeDtypeStruct((N * C,), jnp.float32),
        scratch_types=[
            pltpu.VMEM((KBP,), jnp.int32),
            pltpu.VMEM((KBP,), jnp.int32),
            pltpu.VMEM((KBP,), jnp.float32),
            pltpu.VMEM((KBP,), jnp.float32),
            pltpu.VMEM((K, B, C), jnp.float32),
            pltpu.VMEM((K, B, C), jnp.float32),
            pltpu.VMEM((B * C,), jnp.float32),
            pltpu.SemaphoreType.DMA,
            pltpu.SemaphoreType.DMA,
            pltpu.SemaphoreType.DMA,
        ],
    )(_unpool_body)
    return f(table, w_arr, idx_arr)


def kernel(inputs, weight, nn_index):
    idx32 = nn_index.astype(jnp.int32)
    idx_r = idx32.reshape(NW, NBLK, B, K).transpose(0, 1, 3, 2)
    idx_p = jnp.pad(idx_r, ((0, 0), (0, 0), (0, 0), (0, BP - B)))
    idx_arr = idx_p.reshape(NW * NBLK * KBP)

    w_r = weight.reshape(NW, NBLK, B, K).transpose(0, 1, 3, 2)
    w_p = jnp.pad(w_r, ((0, 0), (0, 0), (0, 0), (0, BP - B)))
    w_arr = w_p.reshape(NW * NBLK * KBP)

    return _unpool(inputs, w_arr, idx_arr).reshape(N, C)


# speedup vs baseline: 1.1820x; 1.1820x over previous
import functools

import jax
import jax.numpy as jnp
from jax import lax
from jax.experimental import pallas as pl
from jax.experimental.pallas import tpu as pltpu
from jax.experimental.pallas import tpu_sc as plsc

M = 25000
N = 100000
K = 3
C = 128

NC = 2
NS = 16
NW = NC * NS
ROWS_PER_W = N // NW
B = 125
BP = 128
NBLK = ROWS_PER_W // B
LANES = 16
CCHUNKS = C // LANES
KBP = K * BP
FULLG = B // LANES
TAIL = B - FULLG * LANES
PIB = lax.GatherScatterMode.PROMISE_IN_BOUNDS
DNUMS = lax.GatherDimensionNumbers(
    offset_dims=(), collapsed_slice_dims=(0,), start_index_map=(0,)
)


def _unpool_body(
    table, w_hbm, idx_hbm, out_hbm,
    idxa, wv0, wv1, rows0, rows1, outv,
    gsem0, gsem1, osem,
):
    wid = lax.axis_index("s") * NC + lax.axis_index("c")
    wv = (wv0, wv1)
    rows = (rows0, rows1)
    gsem = (gsem0, gsem1)

    pltpu.sync_copy(idx_hbm.at[pl.ds(wid * NBLK * KBP, NBLK * KBP)], idxa)

    def prefetch(g, s):
        blkid = wid * NBLK + g
        pltpu.async_copy(w_hbm.at[pl.ds(blkid * KBP, KBP)], wv[s], gsem[s])
        for k in range(K):
            pltpu.async_copy(
                table.at[idxa.at[pl.ds((g * K + k) * BP, B)]],
                rows[s].at[k],
                gsem[s],
            )

    def wait_gathers(s):
        pltpu.make_async_copy(
            w_hbm.at[pl.ds(0, KBP)], wv[s], gsem[s]
        ).wait()
        for k in range(K):
            pltpu.make_async_copy(
                table.at[idxa.at[pl.ds(k * BP, B)]], rows[s].at[k], gsem[s]
            ).wait()

    def wait_out():
        pltpu.make_async_copy(
            outv, out_hbm.at[pl.ds(0, B * C)], osem
        ).wait()

    def fire_out(g):
        pltpu.async_copy(
            outv,
            out_hbm.at[pl.ds((wid * ROWS_PER_W + g * B) * C, B * C)],
            osem,
        )

    def compute(s):
        rs = rows[s]
        ws = wv[s]
        ov = outv

        def make_rowfn(b0, wvecs):
            def rowfn(lane, c1):
                b = b0 + lane
                lv = jnp.full((LANES, 1), lane, dtype=jnp.int32)
                w0 = lax.gather(wvecs[0], lv, DNUMS, (1,), mode=PIB)
                w1 = lax.gather(wvecs[1], lv, DNUMS, (1,), mode=PIB)
                w2 = lax.gather(wvecs[2], lv, DNUMS, (1,), mode=PIB)
                for c in range(CCHUNKS):
                    sl = pl.ds(c * LANES, LANES)
                    ov[pl.ds(b * C + c * LANES, LANES)] = (
                        w0 * rs[0, b, sl]
                        + w1 * rs[1, b, sl]
                        + w2 * rs[2, b, sl]
                    )
                return c1

            return rowfn

        def grp(g16, c2):
            b0 = g16 * LANES
            wvecs = [ws[pl.ds(k * BP + b0, LANES)] for k in range(K)]
            lax.fori_loop(0, LANES, make_rowfn(b0, wvecs), 0, unroll=4)
            return c2

        lax.fori_loop(0, FULLG, grp, 0, unroll=1)
        b0t = FULLG * LANES
        wvecs_t = [ws[pl.ds(k * BP + b0t, LANES)] for k in range(K)]
        lax.fori_loop(0, TAIL, make_rowfn(b0t, wvecs_t), 0, unroll=1)

    prefetch(0, 0)
    prefetch(1, 1)
    wait_gathers(0)
    compute(0)
    fire_out(0)
    prefetch(2, 0)
    wait_out()
    wait_gathers(1)
    compute(1)
    fire_out(1)

    def pair(p, carry):
        g = 2 * p
        prefetch(g + 1, 1)
        wait_out()
        wait_gathers(0)
        compute(0)
        fire_out(g)
        prefetch(g + 2, 0)
        wait_out()
        wait_gathers(1)
        compute(1)
        fire_out(g + 1)
        return carry

    lax.fori_loop(1, NBLK // 2, pair, 0, unroll=1)

    wait_out()
    wait_gathers(0)
    compute(0)
    fire_out(NBLK - 1)
    wait_out()


@jax.jit
def _unpool(table, w_arr, idx_arr):
    mesh = plsc.VectorSubcoreMesh(core_axis_name="c", subcore_axis_name="s")
    f = functools.partial(
        pl.kernel,
        mesh=mesh,
        out_type=jax.ShapeDtypeStruct((N * C,), jnp.float32),
        scratch_types=[
            pltpu.VMEM((NBLK * KBP,), jnp.int32),
            pltpu.VMEM((KBP,), jnp.float32),
            pltpu.VMEM((KBP,), jnp.float32),
            pltpu.VMEM((K, B, C), jnp.float32),
            pltpu.VMEM((K, B, C), jnp.float32),
            pltpu.VMEM((B * C,), jnp.float32),
            pltpu.SemaphoreType.DMA,
            pltpu.SemaphoreType.DMA,
            pltpu.SemaphoreType.DMA,
        ],
    )(_unpool_body)
    return f(table, w_arr, idx_arr)


def kernel(inputs, weight, nn_index):
    idx32 = nn_index.astype(jnp.int32)
    idx_r = idx32.reshape(NW, NBLK, B, K).transpose(0, 1, 3, 2)
    idx_p = jnp.pad(idx_r, ((0, 0), (0, 0), (0, 0), (0, BP - B)))
    idx_arr = idx_p.reshape(NW * NBLK * KBP)

    w_r = weight.reshape(NW, NBLK, B, K).transpose(0, 1, 3, 2)
    w_p = jnp.pad(w_r, ((0, 0), (0, 0), (0, 0), (0, BP - B)))
    w_arr = w_p.reshape(NW * NBLK * KBP)

    return _unpool(inputs, w_arr, idx_arr).reshape(N, C)
